# baseline (speedup 1.0000x reference)
import jax
import jax.numpy as jnp
from jax import lax
from jax.experimental import pallas as pl
from jax.experimental.pallas import tpu as pltpu
from jax.experimental.pallas import tpu_sc as plsc

B, C_IN, L = 1024, 32, 50
N_CAT, VOCAB, EMB = 26, 100000, 32
N_CONT = C_IN - N_CAT
C_OUT = N_CAT * EMB + N_CONT

NC, NS = 2, 16
LB = 2
NBLK = L // LB


def _sc_body(xT, tT, outT, trow, xb0, xb1, ob0, ob1, st, sx0, sx1, so0, so1):
    w = lax.axis_index("s") * NC + lax.axis_index("c")
    xb = (xb0, xb1)
    ob = (ob0, ob1)
    sx = (sx0, sx1)
    so = (so0, so1)

    @pl.when(w < N_CONT)
    def _cont():
        @pl.loop(0, NBLK)
        def _t(t):
            l0 = t * LB
            pltpu.sync_copy(xT.at[pl.ds(l0, LB), N_CAT + w], xb0)
            pltpu.sync_copy(xb0, outT.at[pl.ds(l0, LB), N_CAT * EMB + w])

    @pl.loop(0, N_CAT)
    def _chan(i):
        ch = i * EMB + w

        def block(l0, c, wait_ob, fetch_next):
            xbc, obc = xb[c], ob[c]
            pltpu.make_async_copy(xT.at[pl.ds(l0, LB), i], xbc, sx[c]).wait()
            if wait_ob:
                pltpu.make_async_copy(obc, outT.at[pl.ds(0, LB), ch], so[c]).wait()

            for dl in range(LB):
                @pl.loop(0, B // 16, unroll=8)
                def _k(k, dl=dl):
                    v = xbc[dl, pl.ds(k * 16, 16)]
                    obc[dl, pl.ds(k * 16, 16)] = plsc.load_gather(
                        trow, [v.astype(jnp.int32)]
                    )

            pltpu.make_async_copy(obc, outT.at[pl.ds(l0, LB), ch], so[c]).start()
            if fetch_next:
                pltpu.make_async_copy(
                    xT.at[pl.ds(l0 + 2 * LB, LB), i], xbc, sx[c]
                ).start()

        tcp = pltpu.make_async_copy(tT.at[i, w], trow, st)
        tcp.start()
        pltpu.make_async_copy(xT.at[pl.ds(0, LB), i], xb[0], sx[0]).start()
        pltpu.make_async_copy(xT.at[pl.ds(LB, LB), i], xb[1], sx[1]).start()
        tcp.wait()

        block(0, 0, wait_ob=False, fetch_next=True)
        block(LB, 1, wait_ob=False, fetch_next=True)

        @pl.loop(0, (NBLK - 3) // 2)
        def _pair(p):
            l0 = (2 * p + 2) * LB
            block(l0, 0, wait_ob=True, fetch_next=True)

            xbc, obc = xb[1], ob[1]
            pltpu.make_async_copy(xT.at[pl.ds(l0 + LB, LB), i], xbc, sx[1]).wait()
            pltpu.make_async_copy(obc, outT.at[pl.ds(0, LB), ch], so[1]).wait()
            for dl in range(LB):
                @pl.loop(0, B // 16, unroll=8)
                def _k2(k, dl=dl):
                    v = xbc[dl, pl.ds(k * 16, 16)]
                    obc[dl, pl.ds(k * 16, 16)] = plsc.load_gather(
                        trow, [v.astype(jnp.int32)]
                    )
            pltpu.make_async_copy(obc, outT.at[pl.ds(l0 + LB, LB), ch], so[1]).start()

            @pl.when(p < (NBLK - 3) // 2 - 1)
            def _prefetch():
                pltpu.make_async_copy(
                    xT.at[pl.ds(l0 + 3 * LB, LB), i], xb[1], sx[1]
                ).start()

        block((NBLK - 1) * LB, 0, wait_ob=True, fetch_next=False)

        pltpu.make_async_copy(ob[0], outT.at[pl.ds(0, LB), ch], so[0]).wait()
        pltpu.make_async_copy(ob[1], outT.at[pl.ds(0, LB), ch], so[1]).wait()


@jax.jit
def _multi_embedding(xT, tT):
    mesh = plsc.VectorSubcoreMesh(
        core_axis_name="c", subcore_axis_name="s", num_cores=NC, num_subcores=NS
    )
    return pl.kernel(
        _sc_body,
        out_type=jax.ShapeDtypeStruct((L, C_OUT, B), jnp.float32),
        mesh=mesh,
        scratch_types=[
            pltpu.VMEM((VOCAB,), jnp.float32),
            pltpu.VMEM((LB, B), jnp.float32),
            pltpu.VMEM((LB, B), jnp.float32),
            pltpu.VMEM((LB, B), jnp.float32),
            pltpu.VMEM((LB, B), jnp.float32),
            pltpu.SemaphoreType.DMA,
            pltpu.SemaphoreType.DMA,
            pltpu.SemaphoreType.DMA,
            pltpu.SemaphoreType.DMA,
            pltpu.SemaphoreType.DMA,
        ],
        compiler_params=pltpu.CompilerParams(
            needs_layout_passes=False, use_tc_tiling_on_sc=True
        ),
    )(xT, tT)


def kernel(x, tables):
    xT = jnp.transpose(x, (2, 1, 0))
    tT = jnp.transpose(tables, (0, 2, 1))
    outT = _multi_embedding(xT, tT)
    return jnp.transpose(outT, (2, 1, 0))
